# Initial kernel scaffold; baseline (speedup 1.0000x reference)
import jax
import jax.numpy as jnp
from jax.experimental import pallas as pl


def kernel(emb_e, tokens, table, distribution):
    raise NotImplementedError("write your pallas kernel here")


# speedup vs baseline: 69.1776x; 69.1776x over previous
import functools

import jax
import jax.numpy as jnp
from jax import lax
from jax.experimental import pallas as pl
from jax.experimental.pallas import tpu as pltpu
from jax.experimental.pallas import tpu_sc as plsc

V = 100000
D = 64
B = 4096
L = 200
N_NEG = 5
SPB = L * (N_NEG + 1)

NC = 2
NS = 16
NW = NC * NS
BPW = B // NW

G = 80
N_GATHER = SPB // G
CH = G // 16


def _log_sigmoid(y):
    m = jnp.minimum(y, 0.0)
    t = jnp.exp(-jnp.abs(y))
    z = t / (t + 2.0)
    z2 = z * z
    p = 1.0 + z2 * (1.0 / 3.0 + z2 * (1.0 / 5.0 + z2 * (1.0 / 7.0 + z2 * (1.0 / 9.0))))
    return m - 2.0 * z * p


def _body(emb_hbm, tok_hbm, table_hbm, cdf_hbm, u_hbm,
          loss_hbm, cnt_hbm,
          cdf_v, emb_v, u_v, idx_v, rows_v, lossb_v, cntb_v, sem):
    c = lax.axis_index("c")
    s = lax.axis_index("s")
    wid = s * NC + c
    base_b = wid * BPW

    pltpu.sync_copy(cdf_hbm, cdf_v)
    pltpu.sync_copy(emb_hbm.at[pl.ds(base_b * D, BPW * D)], emb_v)

    lanes = lax.iota(jnp.int32, 16)

    def b_body(bl, carry):
        acc, cnt = carry
        b = base_b + bl
        pltpu.sync_copy(tok_hbm.at[pl.ds(b * L, L)], idx_v.at[pl.ds(0, L)])
        pltpu.sync_copy(u_hbm.at[pl.ds(b * (L * N_NEG), L * N_NEG)],
                        u_v.at[pl.ds(0, L * N_NEG)])

        def s_body(i, _):
            off = pl.multiple_of(i * 16, 16)
            uv = u_v[pl.ds(off, 16)]
            lo = jnp.zeros((16,), jnp.int32)
            hi = jnp.full((16,), V, jnp.int32)
            for _k in range(17):
                mid = lax.shift_right_logical(lo + hi, 1)
                cv = plsc.load_gather(cdf_v, [jnp.minimum(mid, V - 1)])
                pred = cv < uv
                lo = jnp.where(pred, mid + 1, lo)
                hi = jnp.where(pred, hi, mid)
            idx = jnp.minimum(lo, V - 1)
            woff = pl.multiple_of(L + i * 16, 8)
            idx_v[pl.ds(woff, 16)] = idx
            return 0
        lax.fori_loop(0, (L * N_NEG + 15) // 16, s_body, 0)

        def c_body(i, cnt):
            off = pl.multiple_of(i * 16, 16)
            tok = idx_v[pl.ds(off, 16)]
            gpos = lanes + i * 16
            ok = jnp.logical_and(gpos < L, tok != 0)
            return cnt + jnp.where(ok, 1.0, 0.0)
        cnt = lax.fori_loop(0, (L + 15) // 16, c_body, cnt)

        ebase = jnp.full((16,), bl * D, jnp.int32)

        def g_body(g, acc):
            goff = pl.multiple_of(g * G, 16)
            cp = pltpu.async_copy(
                table_hbm.at[idx_v.at[pl.ds(goff, G)]], rows_v, sem)
            cp.wait()
            for ci in range(CH):
                rowi = lanes + ci * 16

                def d_body(d, dot):
                    dv = jnp.full((16,), d, jnp.int32)
                    es = plsc.load_gather(emb_v, [ebase + dv])
                    val = plsc.load_gather(rows_v, [rowi, dv])
                    return dot + es * val
                dot = lax.fori_loop(0, D, d_body,
                                    jnp.zeros((16,), jnp.float32))

                gpos = lanes + (g * G + ci * 16)
                sign = jnp.where(gpos < L, 1.0, -1.0)
                acc = acc + _log_sigmoid(sign * dot)
            return acc
        acc = lax.fori_loop(0, N_GATHER, g_body, acc)
        return (acc, cnt)

    acc0 = jnp.zeros((16,), jnp.float32)
    cnt0 = jnp.zeros((16,), jnp.float32)
    acc, cnt = lax.fori_loop(0, BPW, b_body, (acc0, cnt0))

    lossb_v[...] = acc
    cntb_v[...] = cnt
    pltpu.sync_copy(lossb_v, loss_hbm.at[pl.ds(wid * 16, 16)])
    pltpu.sync_copy(cntb_v, cnt_hbm.at[pl.ds(wid * 16, 16)])


@jax.jit
def kernel(emb_e, tokens, table, distribution):
    cdf = jnp.cumsum(distribution)
    u = jax.random.uniform(jax.random.key(42), (B, L * N_NEG),
                           dtype=jnp.float32, minval=0.0, maxval=cdf[-1])
    tokens = tokens.astype(jnp.int32)

    mesh = plsc.VectorSubcoreMesh(core_axis_name="c", subcore_axis_name="s",
                                  num_cores=NC, num_subcores=NS)
    run = pl.kernel(
        _body,
        out_type=(jax.ShapeDtypeStruct((NW * 16,), jnp.float32),
                  jax.ShapeDtypeStruct((NW * 16,), jnp.float32)),
        mesh=mesh,
        scratch_types=[
            pltpu.VMEM((V,), jnp.float32),
            pltpu.VMEM((BPW * D,), jnp.float32),
            pltpu.VMEM((1008,), jnp.float32),
            pltpu.VMEM((SPB + 8,), jnp.int32),
            pltpu.VMEM((G, D), jnp.float32),
            pltpu.VMEM((16,), jnp.float32),
            pltpu.VMEM((16,), jnp.float32),
            pltpu.SemaphoreType.DMA,
        ],
        compiler_params=pltpu.CompilerParams(needs_layout_passes=False,
                                             use_tc_tiling_on_sc=False),
    )
    loss_parts, cnt_parts = run(emb_e.reshape(-1), tokens.reshape(-1),
                                table, cdf, u.reshape(-1))
    n_token = (N_NEG + 1) * jnp.sum(cnt_parts)
    return -jnp.sum(loss_parts) / n_token


# speedup vs baseline: 86.2768x; 1.2472x over previous
import functools

import jax
import jax.numpy as jnp
from jax import lax
from jax.experimental import pallas as pl
from jax.experimental.pallas import tpu as pltpu
from jax.experimental.pallas import tpu_sc as plsc

V = 100000
D = 64
B = 4096
L = 200
N_NEG = 5
SPB = L * (N_NEG + 1)

NC = 2
NS = 16
NW = NC * NS
BPW = B // NW

G = 120
N_GATHER = SPB // G
CH = (G + 15) // 16


def _log_sigmoid(y):
    m = jnp.minimum(y, 0.0)
    t = jnp.exp(-jnp.abs(y))
    z = t / (t + 2.0)
    z2 = z * z
    p = 1.0 + z2 * (1.0 / 3.0 + z2 * (1.0 / 5.0 + z2 * (1.0 / 7.0 + z2 * (1.0 / 9.0))))
    return m - 2.0 * z * p


def _body(emb_hbm, tok_hbm, table_hbm, cdf_hbm, u_hbm,
          loss_hbm, cnt_hbm,
          cdf_v, emb_v, u_v, idx_v, rows_v, lossb_v, cntb_v, sem, sem2):
    c = lax.axis_index("c")
    s = lax.axis_index("s")
    wid = s * NC + c
    base_b = wid * BPW

    pltpu.sync_copy(cdf_hbm, cdf_v)
    pltpu.sync_copy(emb_hbm.at[pl.ds(base_b * D, BPW * D)], emb_v)

    lanes = lax.iota(jnp.int32, 16)

    def b_body(bl, carry):
        acc, cnt = carry
        b = base_b + bl
        pltpu.sync_copy(tok_hbm.at[pl.ds(b * L, L)], idx_v.at[pl.ds(0, L)])
        pltpu.sync_copy(u_hbm.at[pl.ds(b * (L * N_NEG), L * N_NEG)],
                        u_v.at[pl.ds(0, L * N_NEG)])

        def s_body(i, _):
            off = pl.multiple_of(i * 16, 16)
            uv = u_v[pl.ds(off, 16)]
            lo = jnp.zeros((16,), jnp.int32)
            hi = jnp.full((16,), V, jnp.int32)
            for _k in range(17):
                mid = lax.shift_right_logical(lo + hi, 1)
                cv = plsc.load_gather(cdf_v, [jnp.minimum(mid, V - 1)])
                pred = cv < uv
                lo = jnp.where(pred, mid + 1, lo)
                hi = jnp.where(pred, hi, mid)
            idx = jnp.minimum(lo, V - 1)
            woff = pl.multiple_of(L + i * 16, 8)
            idx_v[pl.ds(woff, 16)] = idx
            return 0
        lax.fori_loop(0, (L * N_NEG + 15) // 16, s_body, 0)

        def c_body(i, cnt):
            off = pl.multiple_of(i * 16, 16)
            tok = idx_v[pl.ds(off, 16)]
            gpos = lanes + i * 16
            ok = jnp.logical_and(gpos < L, tok != 0)
            return cnt + jnp.where(ok, 1.0, 0.0)
        cnt = lax.fori_loop(0, (L + 15) // 16, c_body, cnt)

        ebase = jnp.full((16,), bl * D, jnp.int32)

        def start_gather(g, buf, s):
            goff = pl.multiple_of(g * G, 8)
            pltpu.async_copy(
                table_hbm.at[idx_v.at[pl.ds(goff, G)]], rows_v.at[buf], s)

        def wait_gather(buf, s):
            pltpu.make_async_copy(
                table_hbm.at[idx_v.at[pl.ds(0, G)]], rows_v.at[buf], s).wait()

        zf = jnp.zeros((16,), jnp.float32)
        zi = jnp.zeros((16,), jnp.int32)

        def compute(g, buf, acc):
            rv = rows_v.at[buf]
            for ci in range(CH):
                rowi = jnp.minimum(lanes + ci * 16, G - 1)

                def d_body(_, c):
                    dot, ea, dv = c
                    es = plsc.load_gather(emb_v, [ea])
                    val = plsc.load_gather(rv, [rowi, dv])
                    return (dot + es * val, ea + 1, dv + 1)
                dot, _, _ = lax.fori_loop(0, D, d_body, (zf, ebase, zi),
                                          unroll=16)
                gposl = lanes + ci * 16
                sign = jnp.where(gposl < L - g * G, 1.0, -1.0)
                term = _log_sigmoid(sign * dot)
                if (ci + 1) * 16 > G:
                    term = jnp.where(gposl < G, term, 0.0)
                acc = acc + term
            return acc

        start_gather(0, 0, sem)

        def gp_body(p, acc):
            g0 = p * 2
            wait_gather(0, sem)
            start_gather(g0 + 1, 1, sem2)
            acc = compute(g0, 0, acc)
            wait_gather(1, sem2)

            @pl.when(g0 + 2 < N_GATHER)
            def _():
                start_gather(g0 + 2, 0, sem)
            acc = compute(g0 + 1, 1, acc)
            return acc
        acc = lax.fori_loop(0, N_GATHER // 2, gp_body, acc)
        return (acc, cnt)

    acc0 = jnp.zeros((16,), jnp.float32)
    cnt0 = jnp.zeros((16,), jnp.float32)
    acc, cnt = lax.fori_loop(0, BPW, b_body, (acc0, cnt0))

    lossb_v[...] = acc
    cntb_v[...] = cnt
    pltpu.sync_copy(lossb_v, loss_hbm.at[pl.ds(wid * 16, 16)])
    pltpu.sync_copy(cntb_v, cnt_hbm.at[pl.ds(wid * 16, 16)])


@jax.jit
def kernel(emb_e, tokens, table, distribution):
    cdf = jnp.cumsum(distribution)
    u = jax.random.uniform(jax.random.key(42), (B, L * N_NEG),
                           dtype=jnp.float32, minval=0.0, maxval=cdf[-1])
    tokens = tokens.astype(jnp.int32)

    mesh = plsc.VectorSubcoreMesh(core_axis_name="c", subcore_axis_name="s",
                                  num_cores=NC, num_subcores=NS)
    run = pl.kernel(
        _body,
        out_type=(jax.ShapeDtypeStruct((NW * 16,), jnp.float32),
                  jax.ShapeDtypeStruct((NW * 16,), jnp.float32)),
        mesh=mesh,
        scratch_types=[
            pltpu.VMEM((V,), jnp.float32),
            pltpu.VMEM((BPW * D,), jnp.float32),
            pltpu.VMEM((1008,), jnp.float32),
            pltpu.VMEM((SPB + 8,), jnp.int32),
            pltpu.VMEM((2, G, D), jnp.float32),
            pltpu.VMEM((16,), jnp.float32),
            pltpu.VMEM((16,), jnp.float32),
            pltpu.SemaphoreType.DMA,
            pltpu.SemaphoreType.DMA,
        ],
        compiler_params=pltpu.CompilerParams(needs_layout_passes=False,
                                             use_tc_tiling_on_sc=False),
    )
    loss_parts, cnt_parts = run(emb_e.reshape(-1), tokens.reshape(-1),
                                table, cdf, u.reshape(-1))
    n_token = (N_NEG + 1) * jnp.sum(cnt_parts)
    return -jnp.sum(loss_parts) / n_token


# speedup vs baseline: 100.5686x; 1.1656x over previous
import jax
import jax.numpy as jnp
from jax import lax
from jax.experimental import pallas as pl
from jax.experimental.pallas import tpu as pltpu
from jax.experimental.pallas import tpu_sc as plsc

V = 100000
D = 64
B = 4096
L = 200
N_NEG = 5
SPB = L * (N_NEG + 1)

NC = 2
NS = 16
NW = NC * NS
BPW = B // NW

G = 120
N_GATHER = SPB // G
CH = (G + 15) // 16
NSV = 64


def _log_sigmoid(y):
    m = jnp.minimum(y, 0.0)
    t = jnp.exp(-jnp.abs(y))
    z = t / (t + 2.0)
    z2 = z * z
    p = 1.0 + z2 * (1.0 / 3.0 + z2 * (1.0 / 5.0 + z2 * (1.0 / 7.0 + z2 * (1.0 / 9.0))))
    return m - 2.0 * z * p


def _body(embb_hbm, tok_hbm, table_hbm, cdf_hbm, u_hbm,
          loss_hbm, cnt_hbm,
          cdf_v, ebs_v, u_v, idx_v, rows_v, lossb_v, cntb_v, sem, sem2):
    c = lax.axis_index("c")
    s = lax.axis_index("s")
    wid = s * NC + c
    base_b = wid * BPW

    pltpu.sync_copy(cdf_hbm, cdf_v)

    lanes = lax.iota(jnp.int32, 16)
    zf = jnp.zeros((16,), jnp.float32)
    zi = jnp.zeros((16,), jnp.int32)

    def b_body(bl, carry):
        acc, cnt = carry
        b = base_b + bl
        pltpu.sync_copy(tok_hbm.at[pl.ds(b * L, L)], idx_v.at[pl.ds(0, L)])
        pltpu.sync_copy(u_hbm.at[pl.ds(b * (L * N_NEG), L * N_NEG)],
                        u_v.at[pl.ds(0, L * N_NEG)])
        pltpu.sync_copy(embb_hbm.at[pl.ds(b * (D * 16), D * 16)], ebs_v)

        def s_body(i, _):
            off = pl.multiple_of(i * 16, 16)
            uv = u_v[pl.ds(off, 16)]
            lo = zi
            hi = jnp.full((16,), V, jnp.int32)
            for _k in range(17):
                mid = lax.shift_right_logical(lo + hi, 1)
                cv = plsc.load_gather(cdf_v, [jnp.minimum(mid, V - 1)])
                pred = cv < uv
                lo = jnp.where(pred, mid + 1, lo)
                hi = jnp.where(pred, hi, mid)
            idx = jnp.minimum(lo, V - 1)
            woff = pl.multiple_of(L + i * 16, 8)
            idx_v[pl.ds(woff, 16)] = idx
            return 0
        lax.fori_loop(0, NSV, s_body, 0, unroll=4)

        def c_body(i, cnt):
            off = pl.multiple_of(i * 16, 16)
            tok = idx_v[pl.ds(off, 16)]
            gpos = lanes + i * 16
            ok = jnp.logical_and(gpos < L, tok != 0)
            return cnt + jnp.where(ok, 1.0, 0.0)
        cnt = lax.fori_loop(0, (L + 15) // 16, c_body, cnt)

        def start_gather(g, buf, sm):
            goff = pl.multiple_of(g * G, 8)
            pltpu.async_copy(
                table_hbm.at[idx_v.at[pl.ds(goff, G)]], rows_v.at[buf], sm)

        def wait_gather(buf, sm):
            pltpu.make_async_copy(
                table_hbm.at[idx_v.at[pl.ds(0, G)]], rows_v.at[buf], sm).wait()

        def compute(g, buf, acc):
            rv = rows_v.at[buf]
            rowi = [jnp.minimum(lanes + ci * 16, G - 1) for ci in range(CH)]

            def d_body(d, c):
                dots, dv = c
                eoff = pl.multiple_of(d * 16, 16)
                es = ebs_v[pl.ds(eoff, 16)]
                new = tuple(dots[ci] + es * plsc.load_gather(rv, [rowi[ci], dv])
                            for ci in range(CH))
                return (new, dv + 1)
            dots, _ = lax.fori_loop(0, D, d_body, ((zf,) * CH, zi), unroll=8)

            for ci in range(CH):
                gposl = lanes + ci * 16
                sign = jnp.where(gposl < L - g * G, 1.0, -1.0)
                term = _log_sigmoid(sign * dots[ci])
                if (ci + 1) * 16 > G:
                    term = jnp.where(gposl < G, term, 0.0)
                acc = acc + term
            return acc

        start_gather(0, 0, sem)

        def gp_body(p, acc):
            g0 = p * 2
            wait_gather(0, sem)
            start_gather(g0 + 1, 1, sem2)
            acc = compute(g0, 0, acc)
            wait_gather(1, sem2)

            @pl.when(g0 + 2 < N_GATHER)
            def _():
                start_gather(g0 + 2, 0, sem)
            acc = compute(g0 + 1, 1, acc)
            return acc
        acc = lax.fori_loop(0, N_GATHER // 2, gp_body, acc)
        return (acc, cnt)

    acc, cnt = lax.fori_loop(0, BPW, b_body, (zf, zf))

    lossb_v[...] = acc
    cntb_v[...] = cnt
    pltpu.sync_copy(lossb_v, loss_hbm.at[pl.ds(wid * 16, 16)])
    pltpu.sync_copy(cntb_v, cnt_hbm.at[pl.ds(wid * 16, 16)])


@jax.jit
def kernel(emb_e, tokens, table, distribution):
    cdf = jnp.cumsum(distribution)
    u = jax.random.uniform(jax.random.key(42), (B, L * N_NEG),
                           dtype=jnp.float32, minval=0.0, maxval=cdf[-1])
    tokens = tokens.astype(jnp.int32)
    emb_bcast = jnp.broadcast_to(emb_e[:, :, None], (B, D, 16)).reshape(-1)

    mesh = plsc.VectorSubcoreMesh(core_axis_name="c", subcore_axis_name="s",
                                  num_cores=NC, num_subcores=NS)
    run = pl.kernel(
        _body,
        out_type=(jax.ShapeDtypeStruct((NW * 16,), jnp.float32),
                  jax.ShapeDtypeStruct((NW * 16,), jnp.float32)),
        mesh=mesh,
        scratch_types=[
            pltpu.VMEM((V,), jnp.float32),
            pltpu.VMEM((D * 16,), jnp.float32),
            pltpu.VMEM((NSV * 16,), jnp.float32),
            pltpu.VMEM((L + NSV * 16 + 24,), jnp.int32),
            pltpu.VMEM((2, G, D), jnp.float32),
            pltpu.VMEM((16,), jnp.float32),
            pltpu.VMEM((16,), jnp.float32),
            pltpu.SemaphoreType.DMA,
            pltpu.SemaphoreType.DMA,
        ],
        compiler_params=pltpu.CompilerParams(needs_layout_passes=False,
                                             use_tc_tiling_on_sc=False),
    )
    loss_parts, cnt_parts = run(emb_bcast, tokens.reshape(-1),
                                table, cdf, u.reshape(-1))
    n_token = (N_NEG + 1) * jnp.sum(cnt_parts)
    return -jnp.sum(loss_parts) / n_token


# speedup vs baseline: 246.2390x; 2.4485x over previous
import jax
import jax.numpy as jnp
from jax import lax
from jax.experimental import pallas as pl
from jax.experimental.pallas import tpu as pltpu
from jax.experimental.pallas import tpu_sc as plsc

V = 100000
D = 64
B = 4096
L = 200
N_NEG = 5
SPB = L * (N_NEG + 1)

NC = 2
NS = 16
NW = NC * NS
BPW = B // NW

G = 120
N_GATHER = SPB // G
CH = (G + 15) // 16
NSV = 64


def _log_sigmoid(y):
    m = jnp.minimum(y, 0.0)
    t = jnp.exp(-jnp.abs(y))
    z = t / (t + 2.0)
    z2 = z * z
    p = 1.0 + z2 * (1.0 / 3.0 + z2 * (1.0 / 5.0 + z2 * (1.0 / 7.0 + z2 * (1.0 / 9.0))))
    return m - 2.0 * z * p


def _body(embb_hbm, tok_hbm, table_hbm, cdf_hbm, u_hbm,
          loss_hbm, cnt_hbm,
          cdf_v, ebs_v, u_v, idx_v, rows_v, lossb_v, cntb_v, sem, sem2):
    c = lax.axis_index("c")
    s = lax.axis_index("s")
    wid = s * NC + c
    base_b = wid * BPW

    pltpu.sync_copy(cdf_hbm, cdf_v)

    lanes = lax.iota(jnp.int32, 16)
    zf = jnp.zeros((16,), jnp.float32)
    zi = jnp.zeros((16,), jnp.int32)

    def b_body(bl, carry):
        acc, cnt = carry
        b = base_b + bl
        pltpu.sync_copy(tok_hbm.at[pl.ds(b * L, L)], idx_v.at[pl.ds(0, L)])
        pltpu.sync_copy(u_hbm.at[pl.ds(b * (L * N_NEG), L * N_NEG)],
                        u_v.at[pl.ds(0, L * N_NEG)])
        pltpu.sync_copy(embb_hbm.at[pl.ds(b * D, D)], ebs_v)

        def s_body(i, _):
            off = pl.multiple_of(i * 16, 16)
            uv = u_v[pl.ds(off, 16)]
            lo = zi
            hi = jnp.full((16,), V, jnp.int32)
            for _k in range(17):
                mid = lax.shift_right_logical(lo + hi, 1)
                cv = plsc.load_gather(cdf_v, [jnp.minimum(mid, V - 1)])
                pred = cv < uv
                lo = jnp.where(pred, mid + 1, lo)
                hi = jnp.where(pred, hi, mid)
            idx = jnp.minimum(lo, V - 1)
            woff = pl.multiple_of(L + i * 16, 8)
            idx_v[pl.ds(woff, 16)] = idx
            return 0
        lax.fori_loop(0, NSV, s_body, 0, unroll=4)

        def c_body(i, cnt):
            off = pl.multiple_of(i * 16, 16)
            tok = idx_v[pl.ds(off, 16)]
            gpos = lanes + i * 16
            ok = jnp.logical_and(gpos < L, tok != 0)
            return cnt + jnp.where(ok, 1.0, 0.0)
        cnt = lax.fori_loop(0, (L + 15) // 16, c_body, cnt)

        def start_gather(g, buf, sm):
            goff = pl.multiple_of(g * G, 8)
            pltpu.async_copy(
                table_hbm.at[idx_v.at[pl.ds(goff, G)]], rows_v.at[buf], sm)

        def wait_gather(buf, sm):
            pltpu.make_async_copy(
                table_hbm.at[idx_v.at[pl.ds(0, G)]], rows_v.at[buf], sm).wait()

        def compute(g, buf, acc):
            rv = rows_v.at[buf]
            rowi = [jnp.minimum(lanes + ci * 16, G - 1) for ci in range(CH)]

            def d_body(_, c):
                dots, dp = c
                es = plsc.load_gather(ebs_v, [dp])
                new = tuple(dots[ci] + es * plsc.load_gather(rv, [rowi[ci], dp])
                            for ci in range(CH))
                return (new, (dp + 1) & (D - 1))
            dots, _ = lax.fori_loop(0, D, d_body, ((zf,) * CH, lanes),
                                    unroll=8)

            for ci in range(CH):
                gposl = lanes + ci * 16
                sign = jnp.where(gposl < L - g * G, 1.0, -1.0)
                term = _log_sigmoid(sign * dots[ci])
                if (ci + 1) * 16 > G:
                    term = jnp.where(gposl < G, term, 0.0)
                acc = acc + term
            return acc

        start_gather(0, 0, sem)

        def gp_body(p, acc):
            g0 = p * 2
            wait_gather(0, sem)
            start_gather(g0 + 1, 1, sem2)
            acc = compute(g0, 0, acc)
            wait_gather(1, sem2)

            @pl.when(g0 + 2 < N_GATHER)
            def _():
                start_gather(g0 + 2, 0, sem)
            acc = compute(g0 + 1, 1, acc)
            return acc
        acc = lax.fori_loop(0, N_GATHER // 2, gp_body, acc)
        return (acc, cnt)

    acc, cnt = lax.fori_loop(0, BPW, b_body, (zf, zf))

    lossb_v[...] = acc
    cntb_v[...] = cnt
    pltpu.sync_copy(lossb_v, loss_hbm.at[pl.ds(wid * 16, 16)])
    pltpu.sync_copy(cntb_v, cnt_hbm.at[pl.ds(wid * 16, 16)])


@jax.jit
def kernel(emb_e, tokens, table, distribution):
    cdf = jnp.cumsum(distribution)
    u = jax.random.uniform(jax.random.key(42), (B, L * N_NEG),
                           dtype=jnp.float32, minval=0.0, maxval=cdf[-1])
    tokens = tokens.astype(jnp.int32)

    mesh = plsc.VectorSubcoreMesh(core_axis_name="c", subcore_axis_name="s",
                                  num_cores=NC, num_subcores=NS)
    run = pl.kernel(
        _body,
        out_type=(jax.ShapeDtypeStruct((NW * 16,), jnp.float32),
                  jax.ShapeDtypeStruct((NW * 16,), jnp.float32)),
        mesh=mesh,
        scratch_types=[
            pltpu.VMEM((V,), jnp.float32),
            pltpu.VMEM((D,), jnp.float32),
            pltpu.VMEM((NSV * 16,), jnp.float32),
            pltpu.VMEM((L + NSV * 16 + 24,), jnp.int32),
            pltpu.VMEM((2, G, D), jnp.float32),
            pltpu.VMEM((16,), jnp.float32),
            pltpu.VMEM((16,), jnp.float32),
            pltpu.SemaphoreType.DMA,
            pltpu.SemaphoreType.DMA,
        ],
        compiler_params=pltpu.CompilerParams(needs_layout_passes=False,
                                             use_tc_tiling_on_sc=False),
    )
    loss_parts, cnt_parts = run(emb_e.reshape(-1), tokens.reshape(-1),
                                table, cdf, u.reshape(-1))
    n_token = (N_NEG + 1) * jnp.sum(cnt_parts)
    return -jnp.sum(loss_parts) / n_token


# speedup vs baseline: 272.7454x; 1.1076x over previous
import jax
import jax.numpy as jnp
from jax import lax
from jax.experimental import pallas as pl
from jax.experimental.pallas import tpu as pltpu
from jax.experimental.pallas import tpu_sc as plsc

V = 100000
D = 64
B = 4096
L = 200
N_NEG = 5
NU = L * N_NEG
SPB = L + NU

NC = 2
NS = 16
NW = NC * NS
BPW = B // NW

G = 120
N_GATHER = SPB // G
CH = (G + 15) // 16
NSV = (NU + 15) // 16

EOFF = L + NU + 8
PK = EOFF + D


def _log_sigmoid(y):
    m = jnp.minimum(y, 0.0)
    t = jnp.exp(-jnp.abs(y))
    z = t / (t + 2.0)
    z2 = z * z
    p = 1.0 + z2 * (1.0 / 3.0 + z2 * (1.0 / 5.0 + z2 * (1.0 / 7.0 + z2 * (1.0 / 9.0))))
    return m - 2.0 * z * p


def _body(pk_hbm, table_hbm, cdf_hbm,
          loss_hbm, cnt_hbm,
          cdf_v, pk_v, rows_v, lossb_v, cntb_v,
          sem, sem2, psem0, psem1):
    c = lax.axis_index("c")
    s = lax.axis_index("s")
    wid = s * NC + c
    base_b = wid * BPW

    pltpu.sync_copy(cdf_hbm, cdf_v)

    lanes = lax.iota(jnp.int32, 16)
    zf = jnp.zeros((16,), jnp.float32)

    def pk_fetch(b, buf, psem):
        pltpu.async_copy(pk_hbm.at[pl.ds(b * PK, PK)], pk_v.at[buf], psem)

    pk_fetch(base_b, 0, psem0)

    def one_b(bl, buf, psem, npsem, carry):
        acc, cnt = carry
        pv = pk_v.at[buf]
        pltpu.make_async_copy(
            pk_hbm.at[pl.ds(0, PK)], pv, psem).wait()

        @pl.when(bl + 1 < BPW)
        def _():
            pk_fetch(base_b + bl + 1, 1 - buf, npsem)

        def s_body(i, _):
            off = pl.multiple_of(L + i * 16, 8)
            uv = plsc.bitcast(pk_v[buf, pl.ds(off, 16)], jnp.float32)
            lo = jnp.zeros((16,), jnp.int32)
            hi = jnp.full((16,), V, jnp.int32)
            for _k in range(17):
                mid = lax.shift_right_logical(lo + hi, 1)
                cv = plsc.load_gather(cdf_v, [jnp.minimum(mid, V - 1)])
                pred = cv < uv
                lo = jnp.where(pred, mid + 1, lo)
                hi = jnp.where(pred, hi, mid)
            idx = jnp.minimum(lo, V - 1)
            pk_v[buf, pl.ds(off, 16)] = idx
            return 0
        lax.fori_loop(0, NSV, s_body, 0, unroll=4)

        def c_body(i, cnt):
            off = pl.multiple_of(i * 16, 16)
            tok = pk_v[buf, pl.ds(off, 16)]
            gpos = lanes + i * 16
            ok = jnp.logical_and(gpos < L, tok != 0)
            return cnt + jnp.where(ok, 1.0, 0.0)
        cnt = lax.fori_loop(0, (L + 15) // 16, c_body, cnt)

        def start_gather(g, rbuf, sm):
            goff = pl.multiple_of(g * G, 8)
            pltpu.async_copy(
                table_hbm.at[pk_v.at[buf, pl.ds(goff, G)]],
                rows_v.at[rbuf], sm)

        def wait_gather(rbuf, sm):
            pltpu.make_async_copy(
                table_hbm.at[pk_v.at[buf, pl.ds(0, G)]],
                rows_v.at[rbuf], sm).wait()

        def compute(g, rbuf, acc):
            rv = rows_v.at[rbuf]
            rowi = [jnp.minimum(lanes + ci * 16, G - 1) for ci in range(CH)]

            def d_body(_, cc):
                dots, rot = cc
                es = plsc.bitcast(plsc.load_gather(pv, [rot + EOFF]),
                                  jnp.float32)
                new = tuple(dots[ci] + es * plsc.load_gather(rv, [rowi[ci], rot])
                            for ci in range(CH))
                return (new, (rot + 1) & (D - 1))
            dots, _ = lax.fori_loop(0, D, d_body, ((zf,) * CH, lanes),
                                    unroll=8)

            for ci in range(CH):
                gposl = lanes + ci * 16
                sign = jnp.where(gposl < L - g * G, 1.0, -1.0)
                term = _log_sigmoid(sign * dots[ci])
                if (ci + 1) * 16 > G:
                    term = jnp.where(gposl < G, term, 0.0)
                acc = acc + term
            return acc

        start_gather(0, 0, sem)

        def gp_body(p, acc):
            g0 = p * 2
            wait_gather(0, sem)
            start_gather(g0 + 1, 1, sem2)
            acc = compute(g0, 0, acc)
            wait_gather(1, sem2)

            @pl.when(g0 + 2 < N_GATHER)
            def _():
                start_gather(g0 + 2, 0, sem)
            acc = compute(g0 + 1, 1, acc)
            return acc
        acc = lax.fori_loop(0, N_GATHER // 2, gp_body, acc)
        return (acc, cnt)

    def b2_body(h, carry):
        carry = one_b(2 * h, 0, psem0, psem1, carry)
        carry = one_b(2 * h + 1, 1, psem1, psem0, carry)
        return carry

    acc, cnt = lax.fori_loop(0, BPW // 2, b2_body, (zf, zf))

    lossb_v[...] = acc
    cntb_v[...] = cnt
    pltpu.sync_copy(lossb_v, loss_hbm.at[pl.ds(wid * 16, 16)])
    pltpu.sync_copy(cntb_v, cnt_hbm.at[pl.ds(wid * 16, 16)])


@jax.jit
def kernel(emb_e, tokens, table, distribution):
    cdf = jnp.cumsum(distribution)
    u = jax.random.uniform(jax.random.key(42), (B, NU),
                           dtype=jnp.float32, minval=0.0, maxval=cdf[-1])
    pk = jnp.concatenate([
        tokens.astype(jnp.int32),
        lax.bitcast_convert_type(u, jnp.int32),
        jnp.zeros((B, 8), jnp.int32),
        lax.bitcast_convert_type(emb_e, jnp.int32),
    ], axis=1).reshape(-1)

    mesh = plsc.VectorSubcoreMesh(core_axis_name="c", subcore_axis_name="s",
                                  num_cores=NC, num_subcores=NS)
    run = pl.kernel(
        _body,
        out_type=(jax.ShapeDtypeStruct((NW * 16,), jnp.float32),
                  jax.ShapeDtypeStruct((NW * 16,), jnp.float32)),
        mesh=mesh,
        scratch_types=[
            pltpu.VMEM((V,), jnp.float32),
            pltpu.VMEM((2, PK), jnp.int32),
            pltpu.VMEM((2, G, D), jnp.float32),
            pltpu.VMEM((16,), jnp.float32),
            pltpu.VMEM((16,), jnp.float32),
            pltpu.SemaphoreType.DMA,
            pltpu.SemaphoreType.DMA,
            pltpu.SemaphoreType.DMA,
            pltpu.SemaphoreType.DMA,
        ],
        compiler_params=pltpu.CompilerParams(needs_layout_passes=False,
                                             use_tc_tiling_on_sc=False),
    )
    loss_parts, cnt_parts = run(pk, table, cdf)
    n_token = (N_NEG + 1) * jnp.sum(cnt_parts)
    return -jnp.sum(loss_parts) / n_token


# speedup vs baseline: 331.6207x; 1.2159x over previous
import jax
import jax.numpy as jnp
from jax import lax
from jax.experimental import pallas as pl
from jax.experimental.pallas import tpu as pltpu
from jax.experimental.pallas import tpu_sc as plsc

V = 100000
D = 64
B = 4096
L = 200
N_NEG = 5
NU = L * N_NEG
SPB = L + NU

NC = 2
NS = 16
NW = NC * NS
BPW = B // NW

G = 120
N_GATHER = SPB // G
CH = (G + 15) // 16
NSV = (NU + 15) // 16

EOFF = L + NU + 8
PK = EOFF + D

K = 8192
NKV = K // 16 + 1


def _log_sigmoid(y):
    m = jnp.minimum(y, 0.0)
    t = jnp.exp(-jnp.abs(y))
    z = t / (t + 2.0)
    z2 = z * z
    p = 1.0 + z2 * (1.0 / 3.0 + z2 * (1.0 / 5.0 + z2 * (1.0 / 7.0 + z2 * (1.0 / 9.0))))
    return m - 2.0 * z * p


def _body(pk_hbm, table_hbm, cdf_hbm,
          loss_hbm, cnt_hbm,
          cdf_v, pk_v, rows_v, sbuf_v, lossb_v, cntb_v,
          sem, sem2, psem0, psem1):
    c = lax.axis_index("c")
    s = lax.axis_index("s")
    wid = s * NC + c
    base_b = wid * BPW

    pltpu.sync_copy(cdf_hbm, cdf_v)

    lanes = lax.iota(jnp.int32, 16)
    zf = jnp.zeros((16,), jnp.float32)

    totv = plsc.load_gather(cdf_v, [jnp.full((16,), V - 1, jnp.int32)])
    cell = totv * (1.0 / K)
    rcell = 1.0 / cell

    def sb_body(i, _):
        jv = (lanes + i * 16).astype(jnp.float32)
        gv = jv * cell
        lo = jnp.zeros((16,), jnp.int32)
        hi = jnp.full((16,), V, jnp.int32)
        for _k in range(17):
            mid = lax.shift_right_logical(lo + hi, 1)
            cv = plsc.load_gather(cdf_v, [jnp.minimum(mid, V - 1)])
            pred = cv < gv
            lo = jnp.where(pred, mid + 1, lo)
            hi = jnp.where(pred, hi, mid)
        sbuf_v[pl.ds(pl.multiple_of(i * 16, 16), 16)] = lo
        return 0
    lax.fori_loop(0, NKV, sb_body, 0, unroll=4)

    def pk_fetch(b, buf, psem):
        pltpu.async_copy(pk_hbm.at[pl.ds(b * PK, PK)], pk_v.at[buf], psem)

    pk_fetch(base_b, 0, psem0)

    def one_b(bl, buf, psem, npsem, carry):
        acc, cnt = carry
        pv = pk_v.at[buf]
        pltpu.make_async_copy(
            pk_hbm.at[pl.ds(0, PK)], pv, psem).wait()

        @pl.when(bl + 1 < BPW)
        def _():
            pk_fetch(base_b + bl + 1, 1 - buf, npsem)

        def s_body(i, _):
            off = pl.multiple_of(L + i * 16, 8)
            uv = plsc.bitcast(pk_v[buf, pl.ds(off, 16)], jnp.float32)
            jb = jnp.clip((uv * rcell).astype(jnp.int32), 0, K - 1)
            lo = plsc.load_gather(sbuf_v, [jnp.maximum(jb - 1, 0)])
            hi = plsc.load_gather(sbuf_v, [jb + 2])
            for _k in range(6):
                mid = lax.shift_right_logical(lo + hi, 1)
                cv = plsc.load_gather(cdf_v, [jnp.minimum(mid, V - 1)])
                pred = cv < uv
                lo = jnp.where(pred, mid + 1, lo)
                hi = jnp.where(pred, hi, mid)
            idx = jnp.minimum(lo, V - 1)
            pk_v[buf, pl.ds(off, 16)] = idx
            return 0
        lax.fori_loop(0, NSV, s_body, 0, unroll=8)

        def c_body(i, cnt):
            off = pl.multiple_of(i * 16, 16)
            tok = pk_v[buf, pl.ds(off, 16)]
            gpos = lanes + i * 16
            ok = jnp.logical_and(gpos < L, tok != 0)
            return cnt + jnp.where(ok, 1.0, 0.0)
        cnt = lax.fori_loop(0, (L + 15) // 16, c_body, cnt)

        def start_gather(g, rbuf, sm):
            goff = pl.multiple_of(g * G, 8)
            pltpu.async_copy(
                table_hbm.at[pk_v.at[buf, pl.ds(goff, G)]],
                rows_v.at[rbuf], sm)

        def wait_gather(rbuf, sm):
            pltpu.make_async_copy(
                table_hbm.at[pk_v.at[buf, pl.ds(0, G)]],
                rows_v.at[rbuf], sm).wait()

        def compute(g, rbuf, acc):
            rv = rows_v.at[rbuf]
            rowi = [jnp.minimum(lanes + ci * 16, G - 1) for ci in range(CH)]

            def d_body(_, cc):
                dots, rot = cc
                es = plsc.bitcast(plsc.load_gather(pv, [rot + EOFF]),
                                  jnp.float32)
                new = tuple(dots[ci] + es * plsc.load_gather(rv, [rowi[ci], rot])
                            for ci in range(CH))
                return (new, (rot + 1) & (D - 1))
            dots, _ = lax.fori_loop(0, D, d_body, ((zf,) * CH, lanes),
                                    unroll=8)

            for ci in range(CH):
                gposl = lanes + ci * 16
                sign = jnp.where(gposl < L - g * G, 1.0, -1.0)
                term = _log_sigmoid(sign * dots[ci])
                if (ci + 1) * 16 > G:
                    term = jnp.where(gposl < G, term, 0.0)
                acc = acc + term
            return acc

        start_gather(0, 0, sem)

        def gp_body(p, acc):
            g0 = p * 2
            wait_gather(0, sem)
            start_gather(g0 + 1, 1, sem2)
            acc = compute(g0, 0, acc)
            wait_gather(1, sem2)

            @pl.when(g0 + 2 < N_GATHER)
            def _():
                start_gather(g0 + 2, 0, sem)
            acc = compute(g0 + 1, 1, acc)
            return acc
        acc = lax.fori_loop(0, N_GATHER // 2, gp_body, acc)
        return (acc, cnt)

    def b2_body(h, carry):
        carry = one_b(2 * h, 0, psem0, psem1, carry)
        carry = one_b(2 * h + 1, 1, psem1, psem0, carry)
        return carry

    acc, cnt = lax.fori_loop(0, BPW // 2, b2_body, (zf, zf))

    lossb_v[...] = acc
    cntb_v[...] = cnt
    pltpu.sync_copy(lossb_v, loss_hbm.at[pl.ds(wid * 16, 16)])
    pltpu.sync_copy(cntb_v, cnt_hbm.at[pl.ds(wid * 16, 16)])


@jax.jit
def kernel(emb_e, tokens, table, distribution):
    cdf = jnp.cumsum(distribution)
    u = jax.random.uniform(jax.random.key(42), (B, NU),
                           dtype=jnp.float32, minval=0.0, maxval=cdf[-1])
    pk = jnp.concatenate([
        tokens.astype(jnp.int32),
        lax.bitcast_convert_type(u, jnp.int32),
        jnp.zeros((B, 8), jnp.int32),
        lax.bitcast_convert_type(emb_e, jnp.int32),
    ], axis=1).reshape(-1)

    mesh = plsc.VectorSubcoreMesh(core_axis_name="c", subcore_axis_name="s",
                                  num_cores=NC, num_subcores=NS)
    run = pl.kernel(
        _body,
        out_type=(jax.ShapeDtypeStruct((NW * 16,), jnp.float32),
                  jax.ShapeDtypeStruct((NW * 16,), jnp.float32)),
        mesh=mesh,
        scratch_types=[
            pltpu.VMEM((V,), jnp.float32),
            pltpu.VMEM((2, PK), jnp.int32),
            pltpu.VMEM((2, G, D), jnp.float32),
            pltpu.VMEM((NKV * 16,), jnp.int32),
            pltpu.VMEM((16,), jnp.float32),
            pltpu.VMEM((16,), jnp.float32),
            pltpu.SemaphoreType.DMA,
            pltpu.SemaphoreType.DMA,
            pltpu.SemaphoreType.DMA,
            pltpu.SemaphoreType.DMA,
        ],
        compiler_params=pltpu.CompilerParams(needs_layout_passes=False,
                                             use_tc_tiling_on_sc=False),
    )
    loss_parts, cnt_parts = run(pk, table, cdf)
    n_token = (N_NEG + 1) * jnp.sum(cnt_parts)
    return -jnp.sum(loss_parts) / n_token


# speedup vs baseline: 335.0388x; 1.0103x over previous
import jax
import jax.numpy as jnp
from jax import lax
from jax.experimental import pallas as pl
from jax.experimental.pallas import tpu as pltpu
from jax.experimental.pallas import tpu_sc as plsc

V = 100000
D = 64
B = 4096
L = 200
N_NEG = 5
NU = L * N_NEG
SPB = L + NU

NC = 2
NS = 16
NW = NC * NS
BPW = B // NW

G = 120
N_GATHER = SPB // G
CH = (G + 15) // 16
NSV = (NU + 15) // 16

DW = D // 2
EOFF = L + NU + 8
PK = EOFF + DW
MASKHI = jnp.int32(-65536)

K = 8192
NKV = K // 16 + 1


def _log_sigmoid(y):
    m = jnp.minimum(y, 0.0)
    t = jnp.exp(-jnp.abs(y))
    z = t / (t + 2.0)
    z2 = z * z
    p = 1.0 + z2 * (1.0 / 3.0 + z2 * (1.0 / 5.0 + z2 * (1.0 / 7.0 + z2 * (1.0 / 9.0))))
    return m - 2.0 * z * p


def _body(pk_hbm, table_hbm, cdf_hbm,
          loss_hbm, cnt_hbm,
          cdf_v, pk_v, rows_v, sbuf_v, lossb_v, cntb_v,
          sem, sem2, psem0, psem1):
    c = lax.axis_index("c")
    s = lax.axis_index("s")
    wid = s * NC + c
    base_b = wid * BPW

    pltpu.sync_copy(cdf_hbm, cdf_v)

    lanes = lax.iota(jnp.int32, 16)
    zf = jnp.zeros((16,), jnp.float32)

    totv = plsc.load_gather(cdf_v, [jnp.full((16,), V - 1, jnp.int32)])
    cell = totv * (1.0 / K)
    rcell = 1.0 / cell

    def sb_body(i, _):
        jv = (lanes + i * 16).astype(jnp.float32)
        gv = jv * cell
        lo = jnp.zeros((16,), jnp.int32)
        hi = jnp.full((16,), V, jnp.int32)
        for _k in range(17):
            mid = lax.shift_right_logical(lo + hi, 1)
            cv = plsc.load_gather(cdf_v, [jnp.minimum(mid, V - 1)])
            pred = cv < gv
            lo = jnp.where(pred, mid + 1, lo)
            hi = jnp.where(pred, hi, mid)
        sbuf_v[pl.ds(pl.multiple_of(i * 16, 16), 16)] = lo
        return 0
    lax.fori_loop(0, NKV, sb_body, 0, unroll=4)

    def pk_fetch(b, buf, psem):
        pltpu.async_copy(pk_hbm.at[pl.ds(b * PK, PK)], pk_v.at[buf], psem)

    pk_fetch(base_b, 0, psem0)

    def one_b(bl, buf, psem, npsem, carry):
        acc, cnt = carry
        pv = pk_v.at[buf]
        pltpu.make_async_copy(
            pk_hbm.at[pl.ds(0, PK)], pv, psem).wait()

        @pl.when(bl + 1 < BPW)
        def _():
            pk_fetch(base_b + bl + 1, 1 - buf, npsem)

        def s_body(i, _):
            off = pl.multiple_of(L + i * 16, 8)
            uv = plsc.bitcast(pk_v[buf, pl.ds(off, 16)], jnp.float32)
            jb = jnp.clip((uv * rcell).astype(jnp.int32), 0, K - 1)
            lo = plsc.load_gather(sbuf_v, [jnp.maximum(jb - 1, 0)])
            hi = plsc.load_gather(sbuf_v, [jb + 2])
            for _k in range(6):
                mid = lax.shift_right_logical(lo + hi, 1)
                cv = plsc.load_gather(cdf_v, [jnp.minimum(mid, V - 1)])
                pred = cv < uv
                lo = jnp.where(pred, mid + 1, lo)
                hi = jnp.where(pred, hi, mid)
            idx = jnp.minimum(lo, V - 1)
            pk_v[buf, pl.ds(off, 16)] = idx
            return 0
        lax.fori_loop(0, NSV, s_body, 0, unroll=8)

        def c_body(i, cnt):
            off = pl.multiple_of(i * 16, 16)
            tok = pk_v[buf, pl.ds(off, 16)]
            gpos = lanes + i * 16
            ok = jnp.logical_and(gpos < L, tok != 0)
            return cnt + jnp.where(ok, 1.0, 0.0)
        cnt = lax.fori_loop(0, (L + 15) // 16, c_body, cnt)

        def start_gather(g, rbuf, sm):
            goff = pl.multiple_of(g * G, 8)
            pltpu.async_copy(
                table_hbm.at[pk_v.at[buf, pl.ds(goff, G)]],
                rows_v.at[rbuf], sm)

        def wait_gather(rbuf, sm):
            pltpu.make_async_copy(
                table_hbm.at[pk_v.at[buf, pl.ds(0, G)]],
                rows_v.at[rbuf], sm).wait()

        def compute(g, rbuf, acc):
            rv = rows_v.at[rbuf]
            rowi = [jnp.minimum(lanes + ci * 16, G - 1) for ci in range(CH)]

            def d_body(_, cc):
                dots, rot = cc
                ew = plsc.load_gather(pv, [rot + EOFF])
                eslo = plsc.bitcast(jnp.left_shift(ew, 16), jnp.float32)
                eshi = plsc.bitcast(jnp.bitwise_and(ew, MASKHI), jnp.float32)
                new = []
                for ci in range(CH):
                    vw = plsc.load_gather(rv, [rowi[ci], rot])
                    vlo = plsc.bitcast(jnp.left_shift(vw, 16), jnp.float32)
                    vhi = plsc.bitcast(jnp.bitwise_and(vw, MASKHI),
                                       jnp.float32)
                    new.append(dots[ci] + (eslo * vlo + eshi * vhi))
                return (tuple(new), (rot + 1) & (DW - 1))
            dots, _ = lax.fori_loop(0, DW, d_body, ((zf,) * CH, lanes),
                                    unroll=8)

            for ci in range(CH):
                gposl = lanes + ci * 16
                sign = jnp.where(gposl < L - g * G, 1.0, -1.0)
                term = _log_sigmoid(sign * dots[ci])
                if (ci + 1) * 16 > G:
                    term = jnp.where(gposl < G, term, 0.0)
                acc = acc + term
            return acc

        start_gather(0, 0, sem)

        def gp_body(p, acc):
            g0 = p * 2
            wait_gather(0, sem)
            start_gather(g0 + 1, 1, sem2)
            acc = compute(g0, 0, acc)
            wait_gather(1, sem2)

            @pl.when(g0 + 2 < N_GATHER)
            def _():
                start_gather(g0 + 2, 0, sem)
            acc = compute(g0 + 1, 1, acc)
            return acc
        acc = lax.fori_loop(0, N_GATHER // 2, gp_body, acc)
        return (acc, cnt)

    def b2_body(h, carry):
        carry = one_b(2 * h, 0, psem0, psem1, carry)
        carry = one_b(2 * h + 1, 1, psem1, psem0, carry)
        return carry

    acc, cnt = lax.fori_loop(0, BPW // 2, b2_body, (zf, zf))

    lossb_v[...] = acc
    cntb_v[...] = cnt
    pltpu.sync_copy(lossb_v, loss_hbm.at[pl.ds(wid * 16, 16)])
    pltpu.sync_copy(cntb_v, cnt_hbm.at[pl.ds(wid * 16, 16)])


@jax.jit
def kernel(emb_e, tokens, table, distribution):
    cdf = jnp.cumsum(distribution)
    u = jax.random.uniform(jax.random.key(42), (B, NU),
                           dtype=jnp.float32, minval=0.0, maxval=cdf[-1])
    pk = jnp.concatenate([
        tokens.astype(jnp.int32),
        lax.bitcast_convert_type(u, jnp.int32),
        jnp.zeros((B, 8), jnp.int32),
        lax.bitcast_convert_type(
            emb_e.astype(jnp.bfloat16).reshape(B, DW, 2), jnp.int32),
    ], axis=1).reshape(-1)
    tb = lax.bitcast_convert_type(
        table.astype(jnp.bfloat16).reshape(V, DW, 2), jnp.int32)

    mesh = plsc.VectorSubcoreMesh(core_axis_name="c", subcore_axis_name="s",
                                  num_cores=NC, num_subcores=NS)
    run = pl.kernel(
        _body,
        out_type=(jax.ShapeDtypeStruct((NW * 16,), jnp.float32),
                  jax.ShapeDtypeStruct((NW * 16,), jnp.float32)),
        mesh=mesh,
        scratch_types=[
            pltpu.VMEM((V,), jnp.float32),
            pltpu.VMEM((2, PK), jnp.int32),
            pltpu.VMEM((2, G, DW), jnp.int32),
            pltpu.VMEM((NKV * 16,), jnp.int32),
            pltpu.VMEM((16,), jnp.float32),
            pltpu.VMEM((16,), jnp.float32),
            pltpu.SemaphoreType.DMA,
            pltpu.SemaphoreType.DMA,
            pltpu.SemaphoreType.DMA,
            pltpu.SemaphoreType.DMA,
        ],
        compiler_params=pltpu.CompilerParams(needs_layout_passes=False,
                                             use_tc_tiling_on_sc=False),
    )
    loss_parts, cnt_parts = run(pk, tb, cdf)
    n_token = (N_NEG + 1) * jnp.sum(cnt_parts)
    return -jnp.sum(loss_parts) / n_token
